# baseline (speedup 1.0000x reference)
import jax
import jax.numpy as jnp
from jax import lax
from jax.experimental import pallas as pl
from jax.experimental.pallas import tpu as pltpu

SL = 30


def _topk_body(wrow_ref, wcol_ref, out_ref):
    st = wrow_ref.shape[1]
    wi = jnp.broadcast_to(wrow_ref[...], (st, st))
    wj = jnp.broadcast_to(wcol_ref[...], (st, st))
    jj = lax.broadcasted_iota(jnp.int32, (st, st), 0)
    ii = lax.broadcasted_iota(jnp.int32, (st, st), 1)
    beats = (wj > wi) | ((wj == wi) & (jj < ii))
    rank = jnp.sum(beats.astype(jnp.int32), axis=0, keepdims=True)
    rb = jnp.broadcast_to(rank, (SL, st))
    rr = lax.broadcasted_iota(jnp.int32, (SL, st), 0)
    iidx = lax.broadcasted_iota(jnp.int32, (SL, st), 1)
    out_ref[...] = jnp.sum(jnp.where(rb == rr, iidx, 0), axis=1, keepdims=True)


def _make_row_gather(n, rows):
    def _body(idx_ref, src_ref, out_ref, sem):
        copies = [
            pltpu.make_async_copy(
                src_ref.at[pl.ds(idx_ref[0, i] * rows, rows), :],
                out_ref.at[i],
                sem,
            )
            for i in range(n)
        ]
        for c in copies:
            c.start()
        for c in copies:
            c.wait()

    return _body


def kernel(x, small_x, thw, tem_x, tem_thw, tem_weights, tem_positions,
           tem_indices):
    h, w = 16, 16
    xdim = x.shape[-1]
    t = x.shape[0] // ((h // 2) * (w // 2) * 2 * 2)
    rows_per_frame = x.shape[0] // t
    srows = small_x.shape[0] // t
    st = tem_weights.shape[0]
    K = srows * xdim

    klarge2 = pl.pallas_call(
        _topk_body,
        out_shape=jax.ShapeDtypeStruct((SL, 1), jnp.int32),
    )(tem_weights.reshape(1, st), tem_weights.reshape(st, 1))
    klarge = klarge2.reshape(1, SL)

    cent3 = pl.pallas_call(
        _make_row_gather(SL, srows),
        in_specs=[
            pl.BlockSpec(memory_space=pltpu.SMEM),
            pl.BlockSpec(memory_space=pl.ANY),
        ],
        out_specs=pl.BlockSpec(memory_space=pl.ANY),
        out_shape=jax.ShapeDtypeStruct((SL, srows, xdim), jnp.float32),
        scratch_shapes=[pltpu.SemaphoreType.DMA],
    )(klarge, tem_x)
    centf = cent3.reshape(SL, K)

    sflat = small_x.reshape(t, K)
    NK = 12
    TK = K // NK

    def _dist_body(c_ref, s_ref, o_ref, acc_ref, s2_ref):
        k = pl.program_id(0)

        @pl.when(k == 0)
        def _init():
            acc_ref[...] = jnp.zeros_like(acc_ref)
            s2_ref[...] = jnp.zeros_like(s2_ref)

        c = c_ref[...]
        s = s_ref[...]
        acc_ref[...] += lax.dot_general(
            s, c, (((1,), (1,)), ((), ())),
            preferred_element_type=jnp.float32)
        s2_ref[...] += lax.dot_general(
            s * s, jnp.ones((TK, 1), jnp.float32),
            (((1,), (0,)), ((), ())),
            preferred_element_type=jnp.float32)

        @pl.when(k == NK - 1)
        def _finish():
            score = s2_ref[...] - 2.0 * acc_ref[...]
            m = jnp.min(score, axis=0, keepdims=True)
            ji = lax.broadcasted_iota(jnp.int32, (t, SL), 0)
            big = jnp.where(score == m, ji, jnp.int32(2**30))
            o_ref[...] = jnp.min(big, axis=0, keepdims=True)

    idx2 = pl.pallas_call(
        _dist_body,
        grid=(NK,),
        in_specs=[
            pl.BlockSpec((SL, TK), lambda k: (0, k)),
            pl.BlockSpec((t, TK), lambda k: (0, k)),
        ],
        out_specs=pl.BlockSpec((1, SL), lambda k: (0, 0)),
        out_shape=jax.ShapeDtypeStruct((1, SL), jnp.int32),
        scratch_shapes=[
            pltpu.VMEM((t, SL), jnp.float32),
            pltpu.VMEM((t, 1), jnp.float32),
        ],
    )(centf, sflat)

    spa_x = pl.pallas_call(
        _make_row_gather(SL, rows_per_frame),
        in_specs=[
            pl.BlockSpec(memory_space=pltpu.SMEM),
            pl.BlockSpec(memory_space=pl.ANY),
        ],
        out_specs=pl.BlockSpec(memory_space=pl.ANY),
        out_shape=jax.ShapeDtypeStruct((SL, rows_per_frame, xdim),
                                       jnp.float32),
        scratch_shapes=[pltpu.SemaphoreType.DMA],
    )(idx2, x)

    spa_thw = thw.at[0].set(SL)
    return spa_x, spa_thw, idx2.reshape(SL)
